# baseline (device time: 124307 ns/iter reference)
import jax
import jax.numpy as jnp
from jax import lax
from jax.experimental import pallas as pl
from jax.experimental.pallas import tpu as pltpu

B, S, H, Dh, Dr = 4, 256, 32, 128, 64
D = H * Dh
MB = 128
_VMEM_LIMIT = 100 * 1024 * 1024
_CP = pltpu.CompilerParams(vmem_limit_bytes=_VMEM_LIMIT)
_MESH = pl.DeviceIdType.MESH

NQ_BLK = 8
NQR_BLK = 8
GRID_A = NQ_BLK + NQR_BLK


def _proj_attn(x_batch, Wdkv, Wuk, Wuv, Wkr, Wq, Wqr):
    dc = Wdkv.shape[1]
    scale = (Dh + Dr) ** -0.5

    def body(x_ref, wdkv_ref, wuk_ref, wuv_ref, wkr_ref, wq_ref, wqr_ref,
             o_ref,
             q_scr, kr_scr, k_scr, v_scr,
             c_mine, c_peer, wuk_bf, wuv_bf, wuk_p, wuv_p,
             send_sems, recv_sems):
        j = pl.program_id(0)
        my_x = lax.axis_index("x")
        peer = (1 - my_x, lax.axis_index("y"), lax.axis_index("z"))

        rdma_c = pltpu.make_async_remote_copy(
            src_ref=c_mine, dst_ref=c_peer,
            send_sem=send_sems.at[0], recv_sem=recv_sems.at[0],
            device_id=peer, device_id_type=_MESH)
        rdma_uk = pltpu.make_async_remote_copy(
            src_ref=wuk_bf, dst_ref=wuk_p,
            send_sem=send_sems.at[1], recv_sem=recv_sems.at[1],
            device_id=peer, device_id_type=_MESH)
        rdma_uv = pltpu.make_async_remote_copy(
            src_ref=wuv_bf, dst_ref=wuv_p,
            send_sem=send_sems.at[2], recv_sem=recv_sems.at[2],
            device_id=peer, device_id_type=_MESH)

        @pl.when(j == 0)
        def _start():
            barrier = pltpu.get_barrier_semaphore()
            pl.semaphore_signal(barrier, inc=1, device_id=peer,
                                device_id_type=_MESH)
            pl.semaphore_wait(barrier, 1)
            xb = x_ref[...]
            c_mine[...] = jnp.dot(
                xb, wdkv_ref[...],
                preferred_element_type=jnp.float32).astype(jnp.bfloat16)
            wuk_bf[...] = wuk_ref[...].astype(jnp.bfloat16)
            wuv_bf[...] = wuv_ref[...].astype(jnp.bfloat16)
            rdma_c.start()
            rdma_uk.start()
            rdma_uv.start()
            kr_scr[...] = jnp.dot(xb, wkr_ref[...],
                                  preferred_element_type=jnp.float32)

        x_mine = x_ref[pl.ds(my_x * MB, MB), :]

        @pl.when(j < NQ_BLK)
        def _q():
            chunk = jnp.dot(x_mine, wq_ref[...],
                            preferred_element_type=jnp.float32)
            for i in range(4):
                q_scr[4 * j + i] = chunk[:, i * Dh:(i + 1) * Dh]

        @pl.when(j == 6)
        def _combine():
            rdma_c.wait()
            rdma_uk.wait()
            rdma_uv.wait()
            cm = c_mine[...]
            cp = c_peer[...]
            for g in range(H // 4):
                cs = slice(g * 4 * Dh, (g + 1) * 4 * Dh)
                k4 = (jnp.dot(cm, wuk_bf[:, cs],
                              preferred_element_type=jnp.float32)
                      + jnp.dot(cp, wuk_p[:, cs],
                                preferred_element_type=jnp.float32))
                v4 = (jnp.dot(cm, wuv_bf[:, cs],
                              preferred_element_type=jnp.float32)
                      + jnp.dot(cp, wuv_p[:, cs],
                                preferred_element_type=jnp.float32))
                for i in range(4):
                    k_scr[4 * g + i] = k4[:, i * Dh:(i + 1) * Dh]
                    v_scr[4 * g + i] = v4[:, i * Dh:(i + 1) * Dh]

        @pl.when(j >= NQ_BLK)
        def _attn():
            qr_chunk = jnp.dot(x_mine, wqr_ref[...],
                               preferred_element_type=jnp.float32)
            g = (j - NQ_BLK) * 4
            kr = kr_scr[...]
            dot_t = lambda a, b: lax.dot_general(
                a, b, (((1,), (1,)), ((), ())),
                preferred_element_type=jnp.float32)
            for i in range(4):
                q = q_scr[g + i]
                k = k_scr[g + i]
                v = v_scr[g + i]
                qr = qr_chunk[:, i * Dr:(i + 1) * Dr]
                s = (dot_t(q, k) + dot_t(qr, kr)) * scale
                p = jnp.exp(s)
                p = p / jnp.sum(p, axis=-1, keepdims=True)
                o_ref[:, i * Dh:(i + 1) * Dh] = jnp.dot(
                    p, v, preferred_element_type=jnp.float32)

    full = lambda shape: pl.BlockSpec(shape, lambda j: (0,) * len(shape))
    wq_spec = pl.BlockSpec((D, 512), lambda j: (0, jnp.minimum(j, NQ_BLK - 1)))
    wqr_spec = pl.BlockSpec(
        (D, 256), lambda j: (0, jnp.clip(j - NQ_BLK, 0, NQR_BLK - 1)))

    return pl.pallas_call(
        body,
        grid=(GRID_A,),
        in_specs=[
            full((S, D)),
            full((D, dc)),
            full((dc, D)),
            full((dc, D)),
            full((D, Dr)),
            wq_spec,
            wqr_spec,
        ],
        out_specs=pl.BlockSpec(
            (MB, 4 * Dh), lambda j: (0, jnp.clip(j - NQ_BLK, 0, NQR_BLK - 1))),
        out_shape=jax.ShapeDtypeStruct((MB, D), jnp.float32),
        scratch_shapes=[
            pltpu.VMEM((H, MB, Dh), jnp.float32),
            pltpu.VMEM((S, Dr), jnp.float32),
            pltpu.VMEM((H, S, Dh), jnp.float32),
            pltpu.VMEM((H, S, Dh), jnp.float32),
            pltpu.VMEM((S, dc), jnp.bfloat16),
            pltpu.VMEM((S, dc), jnp.bfloat16),
            pltpu.VMEM((dc, D), jnp.bfloat16),
            pltpu.VMEM((dc, D), jnp.bfloat16),
            pltpu.VMEM((dc, D), jnp.bfloat16),
            pltpu.VMEM((dc, D), jnp.bfloat16),
            pltpu.SemaphoreType.DMA((3,)),
            pltpu.SemaphoreType.DMA((3,)),
        ],
        compiler_params=pltpu.CompilerParams(
            collective_id=0, vmem_limit_bytes=_VMEM_LIMIT,
            dimension_semantics=("arbitrary",)),
    )(x_batch, Wdkv, Wuk, Wuv, Wkr, Wq, Wqr)


NO_BLK = 8


def _out_allgather(O_m, Wo):
    BN = D // NO_BLK
    SPLITS = ((0, 48), (48, 48), (96, 32))

    def body(o_ref, wo_ref, out_ref, scr, stage_c, stage_b,
             csem, bsem, send_sems, recv_sems):
        j = pl.program_id(0)
        bx = lax.axis_index("x")
        by = lax.axis_index("y")
        bz = lax.axis_index("z")
        nbrs = [(1 - bx, by, bz), (bx, 1 - by, bz), (bx, by, 1 - bz)]

        def blk(cx, cy, cz):
            return ((cy * 2 + cz) * 2 + cx) * MB

        me = blk(bx, by, bz)
        x_o = blk(1 - bx, by, bz)
        y_o = blk(bx, 1 - by, bz)
        z_o = blk(bx, by, 1 - bz)
        xy_o = blk(1 - bx, 1 - by, bz)
        xz_o = blk(1 - bx, by, 1 - bz)
        yz_o = blk(bx, 1 - by, 1 - bz)
        anti_o = blk(1 - bx, 1 - by, 1 - bz)

        def chunk_copy(slot, col_off):
            return pltpu.make_async_copy(
                stage_c.at[slot],
                out_ref.at[me // S, pl.ds(me % S, MB), pl.ds(col_off, BN)],
                csem.at[slot])

        def block_copy(slot, row_off):
            return pltpu.make_async_copy(
                stage_b.at[slot],
                out_ref.at[row_off // S, pl.ds(row_off % S, MB), :],
                bsem.at[slot])

        def xfer(link, slot, src_off, n_rows, col_off=0, n_cols=D):
            return pltpu.make_async_remote_copy(
                src_ref=scr.at[pl.ds(src_off, n_rows), pl.ds(col_off, n_cols)],
                dst_ref=scr.at[pl.ds(src_off, n_rows), pl.ds(col_off, n_cols)],
                send_sem=send_sems.at[link, slot],
                recv_sem=recv_sems.at[link, slot],
                device_id=nbrs[link], device_id_type=_MESH)

        @pl.when(j == 0)
        def _barrier():
            barrier = pltpu.get_barrier_semaphore()
            for n in nbrs:
                pl.semaphore_signal(barrier, inc=1, device_id=n,
                                    device_id_type=_MESH)
            pl.semaphore_wait(barrier, 3)

        chunk = jnp.dot(o_ref[...], wo_ref[...],
                        preferred_element_type=jnp.float32)
        slot = lax.rem(j, 2)

        @pl.when(j >= 2)
        def _reuse():
            chunk_copy(slot, (j - 2) * BN).wait()

        stage_c[slot] = chunk
        chunk_copy(slot, j * BN).start()
        scr[pl.ds(me, MB), pl.ds(j * BN, BN)] = chunk.astype(jnp.bfloat16)
        for l in range(3):
            xfer(l, j, me, MB, j * BN, BN).start()

        FWD = ((0, y_o, 1), (1, z_o, 2), (2, x_o, 0))

        def fwd_chunk(jj):
            for out_l, off, in_l in FWD:
                xfer(in_l, jj, off, MB, jj * BN, BN).wait_recv()
                xfer(out_l, NO_BLK + jj, off, MB, jj * BN, BN).start()

        @pl.when(jnp.logical_and(j >= 1, j < NO_BLK - 1))
        def _fwd():
            fwd_chunk(j - 1)

        @pl.when(j == NO_BLK - 1)
        def _tail():
            fwd_chunk(NO_BLK - 2)
            fwd_chunk(NO_BLK - 1)

            def stage_out(offs, base):
                for i, off in enumerate(offs):
                    sl = (base + i) % 2
                    stage_b[sl] = scr[pl.ds(off, MB), :].astype(jnp.float32)
                    block_copy(sl, off).start()

            def wait_out(offs, base):
                for i, off in enumerate(offs):
                    block_copy((base + i) % 2, off).wait()

            stage_out((x_o, y_o), 0)
            wait_out((x_o, y_o), 0)
            stage_out((z_o,), 0)
            for in_l, off in ((0, xy_o), (1, yz_o), (2, xz_o)):
                for jj in range(NO_BLK):
                    xfer(in_l, NO_BLK + jj, off, MB, jj * BN, BN).wait_recv()
            srcs = (yz_o, xz_o, xy_o)
            s3 = [xfer(l, 2 * NO_BLK, srcs[l] + SPLITS[l][0], SPLITS[l][1])
                  for l in range(3)]
            for r in s3:
                r.start()
            wait_out((z_o,), 0)
            stage_out((xy_o, xz_o), 1)
            wait_out((xy_o, xz_o), 1)
            stage_out((yz_o,), 1)
            for r in s3:
                r.wait()
            wait_out((yz_o,), 1)
            stage_out((anti_o,), 0)
            wait_out((anti_o,), 0)
            for l in range(3):
                for jj in range(NO_BLK):
                    xfer(l, jj, me, MB, jj * BN, BN).wait_send()
            for out_l, off, _ in FWD:
                for jj in range(NO_BLK):
                    xfer(out_l, NO_BLK + jj, off, MB,
                         jj * BN, BN).wait_send()
            chunk_copy(0, (NO_BLK - 2) * BN).wait()
            chunk_copy(1, (NO_BLK - 1) * BN).wait()

    return pl.pallas_call(
        body,
        grid=(NO_BLK,),
        in_specs=[
            pl.BlockSpec((MB, D), lambda j: (0, 0)),
            pl.BlockSpec((D, BN), lambda j: (0, j)),
        ],
        out_specs=pl.BlockSpec(memory_space=pltpu.MemorySpace.HBM),
        out_shape=jax.ShapeDtypeStruct((B, S, D), jnp.float32),
        scratch_shapes=[
            pltpu.VMEM((B * S, D), jnp.bfloat16),
            pltpu.VMEM((2, MB, D // NO_BLK), jnp.float32),
            pltpu.VMEM((2, MB, D), jnp.float32),
            pltpu.SemaphoreType.DMA((2,)),
            pltpu.SemaphoreType.DMA((2,)),
            pltpu.SemaphoreType.DMA((3, 2 * NO_BLK + 1)),
            pltpu.SemaphoreType.DMA((3, 2 * NO_BLK + 1)),
        ],
        compiler_params=pltpu.CompilerParams(
            collective_id=1, vmem_limit_bytes=_VMEM_LIMIT,
            dimension_semantics=("arbitrary",)),
    )(O_m, Wo)


def kernel(x, Wdkv, Wuk, Wuv, Wq, Wqr, Wkr, Wo):
    x2 = x.reshape(B * S, D)
    b = lax.axis_index("y") * 2 + lax.axis_index("z")
    x_batch = lax.dynamic_slice(x2, (b * S, 0), (S, D))
    O_m = _proj_attn(x_batch, Wdkv, Wuk, Wuv, Wkr, Wq, Wqr)
    return _out_allgather(O_m, Wo)
